# Initial kernel scaffold; baseline (speedup 1.0000x reference)
import jax
import jax.numpy as jnp
from jax.experimental import pallas as pl


def kernel(x):
    raise NotImplementedError("write your pallas kernel here")


# speedup vs baseline: 3.8931x; 3.8931x over previous
import functools
import math

import jax
import jax.numpy as jnp
from jax import lax
from jax.experimental import pallas as pl
from jax.experimental.pallas import tpu as pltpu
from jax.experimental.pallas import tpu_sc as plsc

L = 16


def _skey(v):
    b = lax.bitcast_convert_type(v, jnp.int32)
    return b ^ (lax.shift_right_arithmetic(b, 31) & jnp.int32(0x7FFFFFFF))


def _make_kernel(B, N, K):
    info = plsc.get_sparse_core_info()
    NC, NS = info.num_cores, info.num_subcores
    NW = NC * NS
    assert B % NW == 0
    rows_per_w = B // NW
    nvec = N // L
    mesh = plsc.VectorSubcoreMesh(core_axis_name="c", subcore_axis_name="s")

    def body(x_hbm, out_hbm, rowbuf, outbuf, hist):
        wid = lax.axis_index("s") * NC + lax.axis_index("c")
        ones = jnp.ones((L,), jnp.int32)
        iota = lax.iota(jnp.int32, L)

        def do_row(i, _):
            row = wid * rows_per_w + i
            pltpu.sync_copy(x_hbm.at[row], rowbuf)

            kk = jnp.int32(K)
            t = jnp.int32(0)
            for lvl in range(4):
                shift = 24 - 8 * lvl

                def zero_body(j, c):
                    hist[pl.ds(j * L, L)] = jnp.zeros((L,), jnp.int32)
                    return c

                lax.fori_loop(0, 256 // L, zero_body, 0)

                def hist_body(j, c, lvl=lvl, shift=shift, t=t):
                    sk = _skey(rowbuf[pl.ds(j * L, L)])
                    if lvl == 0:
                        bucket = lax.shift_right_arithmetic(sk, 24) + 128
                        plsc.addupdate_scatter(hist, [bucket], ones)
                    else:
                        prefix = lax.shift_right_arithmetic(sk, shift + 8)
                        bucket = lax.shift_right_arithmetic(sk, shift) & 255
                        plsc.addupdate_scatter(hist, [bucket], ones,
                                               mask=prefix == t)
                    return c

                lax.fori_loop(0, nvec, hist_body, 0)

                def scan_body(jj, carry, kk=kk):
                    dstar, kprime, tail = carry
                    j = 15 - jj
                    h = hist[pl.ds(j * L, L)]
                    c = plsc.cumsum(h)
                    tot = jnp.sum(h)
                    above = tail + tot - c
                    cond = (above < kk) & (above + h >= kk)
                    dsel = jnp.where(cond, iota + j * L, -1)
                    ksel = jnp.where(cond, kk - above, -1)
                    return (jnp.maximum(dstar, jnp.max(dsel)),
                            jnp.maximum(kprime, jnp.max(ksel)),
                            tail + tot)

                dstar, kprime, _ = lax.fori_loop(
                    0, 256 // L, scan_body,
                    (jnp.int32(-1), jnp.int32(-1), jnp.int32(0)))
                kk = kprime
                t = dstar - 128 if lvl == 0 else (t << 8) | dstar

            tvec = jnp.full((L,), t, jnp.int32)
            thrv = lax.bitcast_convert_type(
                tvec ^ (lax.shift_right_arithmetic(tvec, 31)
                        & jnp.int32(0x7FFFFFFF)), jnp.float32)

            def sum_body(j, acc):
                v = rowbuf[pl.ds(j * L, L)]
                mv = jnp.where(v >= thrv, v, jnp.float32(0))
                outbuf[pl.ds(j * L, L)] = mv
                return acc + mv

            acc = lax.fori_loop(0, nvec, sum_body, jnp.zeros((L,), jnp.float32))
            scale = jnp.full((L,), jnp.float32(N)) / jnp.full(
                (L,), jnp.sum(acc))

            def scale_body(j, c):
                outbuf[pl.ds(j * L, L)] = outbuf[pl.ds(j * L, L)] * scale
                return c

            lax.fori_loop(0, nvec, scale_body, 0)
            pltpu.sync_copy(outbuf, out_hbm.at[row])
            return 0

        lax.fori_loop(0, rows_per_w, do_row, 0)

    return pl.kernel(
        body,
        out_type=jax.ShapeDtypeStruct((B, N), jnp.float32),
        mesh=mesh,
        compiler_params=pltpu.CompilerParams(needs_layout_passes=False),
        scratch_types=[
            pltpu.VMEM((N,), jnp.float32),
            pltpu.VMEM((N,), jnp.float32),
            pltpu.VMEM((256,), jnp.int32),
        ],
    )


@jax.jit
def kernel(x):
    B, N = x.shape
    K = int(math.ceil(0.1 * N))
    return _make_kernel(B, N, K)(x)


# speedup vs baseline: 15.2439x; 3.9156x over previous
import functools
import math

import jax
import jax.numpy as jnp
from jax import lax
from jax.experimental import pallas as pl
from jax.experimental.pallas import tpu as pltpu
from jax.experimental.pallas import tpu_sc as plsc

L = 16


def _skey(v):
    b = lax.bitcast_convert_type(v, jnp.int32)
    return b ^ (lax.shift_right_arithmetic(b, 31) & jnp.int32(0x7FFFFFFF))


def _make_kernel(B, N, K):
    info = plsc.get_sparse_core_info()
    NC, NS = info.num_cores, info.num_subcores
    NW = NC * NS
    assert B % NW == 0
    rows_per_w = B // NW
    nvec = N // L
    mesh = plsc.VectorSubcoreMesh(core_axis_name="c", subcore_axis_name="s")

    def body(x_hbm, out_hbm, rowbuf, outbuf, hist):
        wid = lax.axis_index("s") * NC + lax.axis_index("c")
        ones = jnp.ones((L,), jnp.int32)
        iota = lax.iota(jnp.int32, L)

        def do_row(i, _):
            row = wid * rows_per_w + i
            pltpu.sync_copy(x_hbm.at[row], rowbuf)

            kk = jnp.int32(K)
            t = jnp.int32(0)
            for lvl in range(4):
                shift = 24 - 8 * lvl

                @plsc.parallel_loop(0, 256, step=L, unroll=4)
                def _(j):
                    hist[pl.ds(j, L)] = jnp.zeros((L,), jnp.int32)

                @plsc.parallel_loop(0, N, step=L, unroll=8)
                def _(j, lvl=lvl, shift=shift, t=t):
                    sk = _skey(rowbuf[pl.ds(j, L)])
                    if lvl == 0:
                        bucket = lax.shift_right_arithmetic(sk, 24) + 128
                        plsc.addupdate_scatter(hist, [bucket], ones)
                    else:
                        prefix = lax.shift_right_arithmetic(sk, shift + 8)
                        bucket = lax.shift_right_arithmetic(sk, shift) & 255
                        plsc.addupdate_scatter(hist, [bucket], ones,
                                               mask=prefix == t)

                def scan_body(jj, carry, kk=kk):
                    dstar, kprime, tail = carry
                    j = 15 - jj
                    h = hist[pl.ds(j * L, L)]
                    c = plsc.cumsum(h)
                    tot = jnp.sum(h)
                    above = tail + tot - c
                    cond = (above < kk) & (above + h >= kk)
                    dsel = jnp.where(cond, iota + j * L, -1)
                    ksel = jnp.where(cond, kk - above, -1)
                    return (jnp.maximum(dstar, jnp.max(dsel)),
                            jnp.maximum(kprime, jnp.max(ksel)),
                            tail + tot)

                dstar, kprime, _ = lax.fori_loop(
                    0, 256 // L, scan_body,
                    (jnp.int32(-1), jnp.int32(-1), jnp.int32(0)))
                kk = kprime
                t = dstar - 128 if lvl == 0 else (t << 8) | dstar

            tvec = jnp.full((L,), t, jnp.int32)
            thrv = lax.bitcast_convert_type(
                tvec ^ (lax.shift_right_arithmetic(tvec, 31)
                        & jnp.int32(0x7FFFFFFF)), jnp.float32)

            @plsc.parallel_loop(0, N, step=L, unroll=8,
                                carry=jnp.zeros((L,), jnp.float32))
            def acc(j, acc):
                v = rowbuf[pl.ds(j, L)]
                mv = jnp.where(v >= thrv, v, jnp.float32(0))
                outbuf[pl.ds(j, L)] = mv
                return acc + mv

            scale = jnp.full((L,), jnp.float32(N)) / jnp.full(
                (L,), jnp.sum(acc))

            @plsc.parallel_loop(0, N, step=L, unroll=8)
            def _(j):
                outbuf[pl.ds(j, L)] = outbuf[pl.ds(j, L)] * scale
            pltpu.sync_copy(outbuf, out_hbm.at[row])
            return 0

        lax.fori_loop(0, rows_per_w, do_row, 0)

    return pl.kernel(
        body,
        out_type=jax.ShapeDtypeStruct((B, N), jnp.float32),
        mesh=mesh,
        compiler_params=pltpu.CompilerParams(needs_layout_passes=False),
        scratch_types=[
            pltpu.VMEM((N,), jnp.float32),
            pltpu.VMEM((N,), jnp.float32),
            pltpu.VMEM((256,), jnp.int32),
        ],
    )


@jax.jit
def kernel(x):
    B, N = x.shape
    K = int(math.ceil(0.1 * N))
    return _make_kernel(B, N, K)(x)
